# baseline (device time: 150782 ns/iter reference)
import functools
import os

import jax
import jax.numpy as jnp
from jax import lax
from jax.experimental import pallas as pl
from jax.experimental.pallas import tpu as pltpu

P = 8
M = 1024
K = 8192
N = 4096
NBLK = N // P
KT = 1024
NK = K // KT
NPAIR = P // 2
WB = 2

NO_COMM = bool(os.environ.get("A2A_NO_COMM"))
NO_COMPUTE = bool(os.environ.get("A2A_NO_COMPUTE"))


def kernel(x, w_mat):
    def body(x_ref, w_ref, out_ref, xb, xin, y_send, recv_buf, stage,
             w_buf, wbf, acc, x_sems, w_sems, send_sems, recv_sems,
             stage_sems):
        my = lax.axis_index("i")

        barrier_sem = pltpu.get_barrier_semaphore()
        for d in range(P):
            pl.semaphore_signal(
                barrier_sem, inc=1,
                device_id=(d,), device_id_type=pl.DeviceIdType.MESH,
            )
        pl.semaphore_wait(barrier_sem, P)

        def w_dma(t):
            i, k = divmod(t, NK)
            q = (my // 2 + i) % NPAIR
            return pltpu.make_async_copy(
                w_ref.at[pl.ds(k * KT, KT), pl.ds(q * 2 * NBLK, 2 * NBLK)],
                w_buf.at[t % WB],
                w_sems.at[t % WB],
            )

        def x_dma(k):
            return pltpu.make_async_copy(
                x_ref.at[:, pl.ds(k * KT, KT)], xin.at[k % 2],
                x_sems.at[k % 2],
            )

        def recv_desc(s):
            return pltpu.make_async_remote_copy(
                src_ref=y_send.at[0],
                dst_ref=recv_buf.at[s],
                send_sem=send_sems.at[0],
                recv_sem=recv_sems.at[s],
                device_id=(my,),
                device_id_type=pl.DeviceIdType.MESH,
            )

        def stage_dma(sp, s):
            return pltpu.make_async_copy(
                stage.at[sp],
                out_ref.at[pl.ds(s * M, M), :],
                stage_sems.at[sp],
            )

        if not NO_COMPUTE:
            for t0 in range(WB - 1):
                w_dma(t0).start()
            x_dma(0).start()

        def pair_body(i, carry):
            q = (my // 2 + i) % NPAIR
            j0 = 2 * q
            if not NO_COMPUTE:
                for k in range(NK):
                    t = i * NK + k
                    @pl.when(t + WB - 1 < NPAIR * NK)
                    def _():
                        w_dma(t + WB - 1).start()

                    @pl.when(i == 0)
                    def _():
                        if k + 1 < NK:
                            x_dma(k + 1).start()
                        x_dma(k).wait()
                        xb[:, k * KT:(k + 1) * KT] = xin[k % 2].astype(
                            jnp.bfloat16)
                    w_dma(t).wait()
                    wbf[:, :] = w_buf[t % WB].astype(jnp.bfloat16)
                    for d in (0, 1):
                        part = jnp.dot(
                            xb[:, k * KT:(k + 1) * KT],
                            wbf[:, d * NBLK:(d + 1) * NBLK],
                            preferred_element_type=jnp.float32,
                        )
                        if k == 0:
                            acc[d, :, :] = part
                        else:
                            acc[d, :, :] = acc[d, :, :] + part
                for d in (0, 1):
                    y = acc[d]
                    y = y * (1.0 / (1.0 + jnp.exp(-y)))
                    y_send[2 * i + d, :, :] = y.astype(jnp.bfloat16)
            if not NO_COMM:
                for d in (0, 1):
                    pltpu.make_async_remote_copy(
                        src_ref=y_send.at[2 * i + d],
                        dst_ref=recv_buf.at[my],
                        send_sem=send_sems.at[2 * i + d],
                        recv_sem=recv_sems.at[my],
                        device_id=(j0 + d,),
                        device_id_type=pl.DeviceIdType.MESH,
                    ).start()
                for sp in (0, 1):
                    s = 2 * ((my // 2 - (i - 1)) % NPAIR) + sp

                    @pl.when(i >= 1)
                    def _():
                        recv_desc(s).wait_recv()

                    @pl.when(i >= 2)
                    def _():
                        stage_dma(sp, 0).wait()

                    @pl.when(i >= 1)
                    def _():
                        stage[sp, :, :] = recv_buf[s].astype(jnp.float32)
                        stage_dma(sp, s).start()
            return carry

        lax.fori_loop(0, NPAIR, pair_body, 0)

        if not NO_COMM:
            for sp in (0, 1):
                s = 2 * ((my // 2 - (NPAIR - 1)) % NPAIR) + sp
                recv_desc(s).wait_recv()
                stage_dma(sp, 0).wait()
                stage[sp, :, :] = recv_buf[s].astype(jnp.float32)
                stage_dma(sp, s).start()
            for sp in (0, 1):
                stage_dma(sp, 0).wait()
            for s in range(P):
                pltpu.make_async_remote_copy(
                    src_ref=y_send.at[s],
                    dst_ref=recv_buf.at[s],
                    send_sem=send_sems.at[s],
                    recv_sem=recv_sems.at[s],
                    device_id=(my,),
                    device_id_type=pl.DeviceIdType.MESH,
                ).wait_send()

        @functools.partial(
            pl.run_scoped, sem2=pltpu.SemaphoreType.REGULAR)
        def _(sem2):
            for d in range(P):
                pl.semaphore_signal(
                    sem2, inc=1,
                    device_id=(d,), device_id_type=pl.DeviceIdType.MESH,
                )
            pl.semaphore_wait(sem2, P)

    return pl.pallas_call(
        body,
        out_shape=jax.ShapeDtypeStruct((P * M, NBLK), jnp.float32),
        in_specs=[
            pl.BlockSpec(memory_space=pl.ANY),
            pl.BlockSpec(memory_space=pl.ANY),
        ],
        out_specs=pl.BlockSpec(memory_space=pl.ANY),
        scratch_shapes=[
            pltpu.VMEM((M, K), jnp.bfloat16),
            pltpu.VMEM((2, M, KT), jnp.float32),
            pltpu.VMEM((P, M, NBLK), jnp.bfloat16),
            pltpu.VMEM((P, M, NBLK), jnp.bfloat16),
            pltpu.VMEM((2, M, NBLK), jnp.float32),
            pltpu.VMEM((WB, KT, 2 * NBLK), jnp.float32),
            pltpu.VMEM((KT, 2 * NBLK), jnp.bfloat16),
            pltpu.VMEM((2, M, NBLK), jnp.float32),
            pltpu.SemaphoreType.DMA((2,)),
            pltpu.SemaphoreType.DMA((WB,)),
            pltpu.SemaphoreType.DMA((P,)),
            pltpu.SemaphoreType.DMA((P,)),
            pltpu.SemaphoreType.DMA((2,)),
        ],
        compiler_params=pltpu.CompilerParams(
            collective_id=0,
            vmem_limit_bytes=64 * 1024 * 1024,
        ),
    )(x, w_mat)


# device time: 136706 ns/iter; 1.1030x vs baseline; 1.1030x over previous
import functools
import os

import jax
import jax.numpy as jnp
from jax import lax
from jax.experimental import pallas as pl
from jax.experimental.pallas import tpu as pltpu

P = 8
M = 1024
K = 8192
N = 4096
NBLK = N // P
KT = 1024
NK = K // KT
NPAIR = P // 2
WB = 2

NO_COMM = bool(os.environ.get("A2A_NO_COMM"))
NO_COMPUTE = bool(os.environ.get("A2A_NO_COMPUTE"))


def kernel(x, w_mat):
    def body(x_ref, w_ref, out_ref, xb, xin, y_send, recv_buf, stage,
             w_buf, wbf, acc, x_sems, w_sems, send_sems, recv_sems,
             stage_sems):
        my = lax.axis_index("i")

        barrier_sem = pltpu.get_barrier_semaphore()
        for d in range(P):
            pl.semaphore_signal(
                barrier_sem, inc=1,
                device_id=(d,), device_id_type=pl.DeviceIdType.MESH,
            )
        pl.semaphore_wait(barrier_sem, P)

        def w_dma(t):
            i, k = divmod(t, NK)
            q = (my // 2 + i) % NPAIR
            return pltpu.make_async_copy(
                w_ref.at[pl.ds(k * KT, KT), pl.ds(q * 2 * NBLK, 2 * NBLK)],
                w_buf.at[t % WB],
                w_sems.at[t % WB],
            )

        def x_dma(k):
            return pltpu.make_async_copy(
                x_ref.at[:, pl.ds(k * KT, KT)], xin.at[k % 2],
                x_sems.at[k % 2],
            )

        def recv_desc(s):
            return pltpu.make_async_remote_copy(
                src_ref=y_send.at[0],
                dst_ref=recv_buf.at[s],
                send_sem=send_sems.at[0],
                recv_sem=recv_sems.at[s],
                device_id=(my,),
                device_id_type=pl.DeviceIdType.MESH,
            )

        def stage_dma(sp, s):
            return pltpu.make_async_copy(
                stage.at[sp],
                out_ref.at[pl.ds(s * M, M), :],
                stage_sems.at[sp],
            )

        if not NO_COMPUTE:
            for t0 in range(WB - 1):
                w_dma(t0).start()
            x_dma(0).start()

        def pair_body(i, carry):
            q = (my // 2 + i) % NPAIR
            j0 = 2 * q
            if not NO_COMPUTE:
                for k in range(NK):
                    t = i * NK + k
                    @pl.when(t + WB - 1 < NPAIR * NK)
                    def _():
                        w_dma(t + WB - 1).start()

                    @pl.when(i == 0)
                    def _():
                        if k + 1 < NK:
                            x_dma(k + 1).start()
                        x_dma(k).wait()
                        xb[:, k * KT:(k + 1) * KT] = xin[k % 2].astype(
                            jnp.bfloat16)
                    w_dma(t).wait()
                    wbf[:, :] = w_buf[t % WB].astype(jnp.bfloat16)
                    for d in (0, 1):
                        part = jnp.dot(
                            xb[:, k * KT:(k + 1) * KT],
                            wbf[:, d * NBLK:(d + 1) * NBLK],
                            preferred_element_type=jnp.float32,
                        )
                        if k == 0:
                            acc[d, :, :] = part
                        else:
                            acc[d, :, :] = acc[d, :, :] + part
                for d in (0, 1):
                    y = acc[d]
                    y = y * (1.0 / (1.0 + jnp.exp(-y)))
                    y_send[2 * i + d, :, :] = y.astype(jnp.bfloat16)
            if not NO_COMM:
                for d in (0, 1):
                    pltpu.make_async_remote_copy(
                        src_ref=y_send.at[2 * i + d],
                        dst_ref=recv_buf.at[my],
                        send_sem=send_sems.at[2 * i + d],
                        recv_sem=recv_sems.at[my],
                        device_id=(j0 + d,),
                        device_id_type=pl.DeviceIdType.MESH,
                    ).start()
            return carry

        lax.fori_loop(0, NPAIR, pair_body, 0)

        if not NO_COMM:
            for w in range(NPAIR):
                for sp in (0, 1):
                    s = 2 * ((my // 2 - w) % NPAIR) + sp
                    recv_desc(s).wait_recv()
                    if w >= 1:
                        stage_dma(sp, 0).wait()
                    stage[sp, :, :] = recv_buf[s].astype(jnp.float32)
                    stage_dma(sp, s).start()
            for sp in (0, 1):
                stage_dma(sp, 0).wait()
            for s in range(P):
                pltpu.make_async_remote_copy(
                    src_ref=y_send.at[s],
                    dst_ref=recv_buf.at[s],
                    send_sem=send_sems.at[s],
                    recv_sem=recv_sems.at[s],
                    device_id=(my,),
                    device_id_type=pl.DeviceIdType.MESH,
                ).wait_send()

        @functools.partial(
            pl.run_scoped, sem2=pltpu.SemaphoreType.REGULAR)
        def _(sem2):
            for d in range(P):
                pl.semaphore_signal(
                    sem2, inc=1,
                    device_id=(d,), device_id_type=pl.DeviceIdType.MESH,
                )
            pl.semaphore_wait(sem2, P)

    return pl.pallas_call(
        body,
        out_shape=jax.ShapeDtypeStruct((P * M, NBLK), jnp.float32),
        in_specs=[
            pl.BlockSpec(memory_space=pl.ANY),
            pl.BlockSpec(memory_space=pl.ANY),
        ],
        out_specs=pl.BlockSpec(memory_space=pl.ANY),
        scratch_shapes=[
            pltpu.VMEM((M, K), jnp.bfloat16),
            pltpu.VMEM((2, M, KT), jnp.float32),
            pltpu.VMEM((P, M, NBLK), jnp.bfloat16),
            pltpu.VMEM((P, M, NBLK), jnp.bfloat16),
            pltpu.VMEM((2, M, NBLK), jnp.float32),
            pltpu.VMEM((WB, KT, 2 * NBLK), jnp.float32),
            pltpu.VMEM((KT, 2 * NBLK), jnp.bfloat16),
            pltpu.VMEM((2, M, NBLK), jnp.float32),
            pltpu.SemaphoreType.DMA((2,)),
            pltpu.SemaphoreType.DMA((WB,)),
            pltpu.SemaphoreType.DMA((P,)),
            pltpu.SemaphoreType.DMA((P,)),
            pltpu.SemaphoreType.DMA((2,)),
        ],
        compiler_params=pltpu.CompilerParams(
            collective_id=0,
            vmem_limit_bytes=64 * 1024 * 1024,
        ),
    )(x, w_mat)
